# baseline (speedup 1.0000x reference)
import functools

import jax
import jax.numpy as jnp
from jax import lax
from jax.experimental import pallas as pl
from jax.experimental.pallas import tpu as pltpu
from jax.experimental.pallas import tpu_sc as plsc

_NW = 32
_B = 160
_L = 16


def kernel(X):
    M, N = X.shape
    nblocks = M // _B
    iters = -(-nblocks // _NW)
    mesh = plsc.VectorSubcoreMesh(core_axis_name="c", subcore_axis_name="s")

    @functools.partial(
        pl.kernel,
        mesh=mesh,
        out_type=[
            jax.ShapeDtypeStruct((M, 160), X.dtype),
            jax.ShapeDtypeStruct((M, 160), X.dtype),
        ],
        scratch_types=[
            pltpu.VMEM((_B, 256), jnp.float32),
            pltpu.VMEM((_B, 160), jnp.float32),
            pltpu.VMEM((_B, 160), jnp.float32),
            pltpu.SemaphoreType.DMA,
            pltpu.SemaphoreType.DMA,
        ],
    )
    def run(x_hbm, p0_hbm, p1_hbm, bufX, bufP0, bufP1, s_in, s_out):
        wid = lax.axis_index("s") * 2 + lax.axis_index("c")

        def blk(i):
            return wid + i * _NW

        def rows_of(b):
            return pl.ds(b * _B, _B)

        def in_copy(b):
            return pltpu.make_async_copy(x_hbm.at[rows_of(b)], bufX, s_in)

        def out_copies(b):
            rows = rows_of(b)
            return (
                pltpu.make_async_copy(bufP0, p0_hbm.at[rows], s_out),
                pltpu.make_async_copy(bufP1, p1_hbm.at[rows], s_out),
            )

        def compute():
            @plsc.parallel_loop(0, _B, 1, unroll=8)
            def _rot(row):
                v = [bufX[row, pl.ds(k * _L, _L)] for k in range(16)]
                for k in range(10):
                    bufP0[row, pl.ds(k * _L, _L)] = v[k]
                for k in range(10):
                    bufP1[row, pl.ds(k * _L, _L)] = v[6 + k]

        def guarded(i, f):
            b = blk(i)

            @pl.when(jnp.logical_and(i >= 0, b < nblocks))
            def _():
                f(b)

        def body(i, carry):
            guarded(i, lambda b: in_copy(b).wait())
            guarded(i - 1, lambda b: [c.wait() for c in out_copies(b)])
            guarded(i, lambda b: compute())
            guarded(i, lambda b: [c.start() for c in out_copies(b)])
            guarded(i + 1, lambda b: in_copy(b).start())
            return carry

        guarded(0, lambda b: in_copy(b).start())
        lax.fori_loop(0, iters, body, 0)
        guarded(iters - 1, lambda b: [c.wait() for c in out_copies(b)])

    p0, p1 = run(X)
    return (p0, p1)


# speedup vs baseline: 1.0406x; 1.0406x over previous
import functools

import jax
import jax.numpy as jnp
from jax import lax
from jax.experimental import pallas as pl
from jax.experimental.pallas import tpu as pltpu
from jax.experimental.pallas import tpu_sc as plsc

_NW = 32
_B = 80
_L = 16


def kernel(X):
    M, N = X.shape
    nblocks = M // _B
    iters = -(-nblocks // _NW)
    mesh = plsc.VectorSubcoreMesh(core_axis_name="c", subcore_axis_name="s")

    @functools.partial(
        pl.kernel,
        mesh=mesh,
        out_type=[
            jax.ShapeDtypeStruct((M, 160), X.dtype),
            jax.ShapeDtypeStruct((M, 160), X.dtype),
        ],
        scratch_types=[
            pltpu.VMEM((2, _B, 256), jnp.float32),
            pltpu.VMEM((2, _B, 160), jnp.float32),
            pltpu.VMEM((2, _B, 160), jnp.float32),
            pltpu.SemaphoreType.DMA,
            pltpu.SemaphoreType.DMA,
            pltpu.SemaphoreType.DMA,
            pltpu.SemaphoreType.DMA,
        ],
    )
    def run(x_hbm, p0_hbm, p1_hbm, bufX, bufP0, bufP1,
            s_in0, s_in1, s_out0, s_out1):
        wid = lax.axis_index("s") * 2 + lax.axis_index("c")

        def blk(i):
            return wid + i * _NW

        def rows_of(b):
            return pl.ds(b * _B, _B)

        def in_copy(slot, b):
            sem = s_in1 if slot else s_in0
            return pltpu.make_async_copy(
                x_hbm.at[rows_of(b)], bufX.at[slot], sem)

        def out_copies(slot, b):
            sem = s_out1 if slot else s_out0
            rows = rows_of(b)
            return (
                pltpu.make_async_copy(bufP0.at[slot], p0_hbm.at[rows], sem),
                pltpu.make_async_copy(bufP1.at[slot], p1_hbm.at[rows], sem),
            )

        def compute(slot, b):
            @plsc.parallel_loop(0, _B, 1, unroll=8)
            def _rot(row):
                v = [bufX[slot, row, pl.ds(k * _L, _L)] for k in range(16)]
                for k in range(10):
                    bufP0[slot, row, pl.ds(k * _L, _L)] = v[k]
                for k in range(10):
                    bufP1[slot, row, pl.ds(k * _L, _L)] = v[6 + k]

        def guarded(i, slot, f):
            b = blk(i)

            @pl.when(jnp.logical_and(i >= 0, b < nblocks))
            def _():
                f(slot, b)

        def body(k, carry):
            for par in (0, 1):
                i = 2 * k + par
                guarded(i, par, lambda s, b: in_copy(s, b).wait())
                guarded(i - 2, par,
                        lambda s, b: [c.wait() for c in out_copies(s, b)])
                guarded(i, par, compute)
                guarded(i, par,
                        lambda s, b: [c.start() for c in out_copies(s, b)])
                guarded(i + 2, par, lambda s, b: in_copy(s, b).start())
            return carry

        guarded(0, 0, lambda s, b: in_copy(s, b).start())
        guarded(1, 1, lambda s, b: in_copy(s, b).start())
        lax.fori_loop(0, iters // 2, body, 0)
        guarded(iters - 2, (iters - 2) % 2,
                lambda s, b: [c.wait() for c in out_copies(s, b)])
        guarded(iters - 1, (iters - 1) % 2,
                lambda s, b: [c.wait() for c in out_copies(s, b)])

    p0, p1 = run(X)
    return (p0, p1)
